# Initial kernel scaffold; baseline (speedup 1.0000x reference)
import jax
import jax.numpy as jnp
from jax.experimental import pallas as pl


def kernel(input, matric, length, target):
    raise NotImplementedError("write your pallas kernel here")


# speedup vs baseline: 3.1248x; 3.1248x over previous
import functools
import math

import jax
import jax.numpy as jnp
from jax import lax
from jax.experimental import pallas as pl
from jax.experimental.pallas import tpu as pltpu
from jax.experimental.pallas import tpu_sc as plsc

BETA = 0.1
PAD = 0
LOG1MBETA = math.log(1.0 - BETA)

NC = 2
NS = 16
L = 16
NW = NC * NS

LN2 = math.log(2.0)
SQRT2 = math.sqrt(2.0)


def _log_vec(s):
    bits = lax.bitcast_convert_type(s, jnp.int32)
    e = lax.shift_right_logical(bits, 23) - 127
    m = lax.bitcast_convert_type(
        (bits & jnp.int32(0x007FFFFF)) | jnp.int32(0x3F800000), jnp.float32)
    adj = m >= SQRT2
    m = jnp.where(adj, 0.5 * m, m)
    e = e + jnp.where(adj, 1, 0)
    t = (m - 1.0) / (m + 1.0)
    t2 = t * t
    lm = 2.0 * t * (1.0 + t2 * (1.0 / 3.0 + t2 * (0.2 + t2 * (1.0 / 7.0))))
    return e.astype(jnp.float32) * LN2 + lm


def _make_sc_kernel(N, T, V):
    rows_per_w = N // NW
    blk = 80
    nblk = rows_per_w // blk
    ngrp = blk // L
    nvec = V // L

    mesh = plsc.VectorSubcoreMesh(
        core_axis_name="c", subcore_axis_name="s", num_cores=NC, num_subcores=NS)

    @functools.partial(
        pl.kernel,
        out_type=(
            jax.ShapeDtypeStruct((NW, L), jnp.float32),
            jax.ShapeDtypeStruct((NW, L), jnp.float32),
        ),
        mesh=mesh,
        compiler_params=pltpu.CompilerParams(needs_layout_passes=False),
        scratch_types=[
            pltpu.VMEM((rows_per_w,), jnp.int32),
            pltpu.VMEM((128,), jnp.float32),
            pltpu.VMEM((blk,), jnp.int32),
            pltpu.VMEM((blk, V), jnp.float32),
            pltpu.VMEM((blk, V), jnp.float32),
            pltpu.VMEM((blk, L), jnp.float32),
            pltpu.VMEM((blk, L), jnp.float32),
            pltpu.VMEM((blk, L), jnp.float32),
            pltpu.VMEM((blk, L), jnp.float32),
            pltpu.VMEM((L,), jnp.float32),
            pltpu.VMEM((L,), jnp.float32),
            pltpu.SemaphoreType.DMA,
            pltpu.SemaphoreType.DMA,
        ],
    )
    def sc_kernel(inp_hbm, mat_hbm, len_hbm, tgt_hbm, loss_out, cnt_out,
                  tgt_v, len_v, idx_v, x_v, m_v,
                  xmax_v, se_v, mdx_v, ms_v, loss_acc, cnt_acc,
                  sem_x, sem_m):
        wid = lax.axis_index("s") * NC + lax.axis_index("c")
        wbase = wid * rows_per_w

        pltpu.sync_copy(tgt_hbm.at[pl.ds(wbase, rows_per_w)], tgt_v)
        pltpu.sync_copy(len_hbm, len_v)

        zeros = jnp.zeros((L,), jnp.float32)
        loss_acc[...] = zeros
        cnt_acc[...] = zeros

        iota = lax.iota(jnp.int32, L)

        def block_body(b, _):
            base_l = b * blk
            base_g = wbase + base_l

            for g in range(ngrp):
                pvec = base_l + g * L + iota
                cur = tgt_v[pl.ds(base_l + g * L, L)]
                prev = plsc.load_gather(tgt_v, [jnp.maximum(pvec - 1, 0)])
                forth = jnp.where(pvec % T == 0, 0, prev)
                idx_v[pl.ds(g * L, L)] = forth * V + cur

            cm = pltpu.async_copy(mat_hbm.at[idx_v], m_v, sem_m)
            cx = pltpu.async_copy(inp_hbm.at[pl.ds(base_g, blk)], x_v, sem_x)
            cm.wait()
            cx.wait()

            def row_body(r, _):
                xv = [x_v[r, pl.ds(16 * j, L)] for j in range(nvec)]
                mx = xv[0]
                for j in range(1, nvec):
                    mx = jnp.maximum(mx, xv[j])
                xmax = jnp.max(mx)
                se = jnp.exp(xv[0] - xmax)
                mv = m_v[r, pl.ds(0, L)]
                mdx = mv * xv[0]
                ms = mv
                for j in range(1, nvec):
                    se = se + jnp.exp(xv[j] - xmax)
                    mv = m_v[r, pl.ds(16 * j, L)]
                    mdx = mdx + mv * xv[j]
                    ms = ms + mv
                xmax_v[r, :] = jnp.broadcast_to(xmax, (L,))
                se_v[r, :] = jnp.broadcast_to(jnp.sum(se), (L,))
                mdx_v[r, :] = jnp.broadcast_to(jnp.sum(mdx), (L,))
                ms_v[r, :] = jnp.broadcast_to(jnp.sum(ms), (L,))
                return 0

            lax.fori_loop(0, blk, row_body, 0, unroll=2)

            for g in range(ngrp):
                pvec = base_l + g * L + iota
                cur = tgt_v[pl.ds(base_l + g * L, L)]
                lrows = g * L + iota
                xt = plsc.load_gather(x_v, [lrows, cur])
                mt = plsc.load_gather(m_v, [lrows, cur])
                zero16 = jnp.zeros((L,), jnp.int32)
                xmax = plsc.load_gather(xmax_v, [lrows, zero16])
                se = plsc.load_gather(se_v, [lrows, zero16])
                lse = xmax + _log_vec(se)
                mdx = plsc.load_gather(mdx_v, [lrows, zero16])
                ms = plsc.load_gather(ms_v, [lrows, zero16])
                bidx = wid * (rows_per_w // T) + pvec // T
                lens = plsc.load_gather(len_v, [bidx])
                s = 1.0 - jnp.exp(LOG1MBETA / lens)
                c = s * (mdx - ms * lse) + (1.0 - s * ms - s * mt) * (xt - lse)
                ispad = cur == PAD
                loss_acc[...] = loss_acc[...] + jnp.where(ispad, 0.0, -c)
                cnt_acc[...] = cnt_acc[...] + jnp.where(ispad, 1.0, 0.0)
            return 0

        lax.fori_loop(0, nblk, block_body, 0)

        pltpu.sync_copy(loss_acc, loss_out.at[wid])
        pltpu.sync_copy(cnt_acc, cnt_out.at[wid])

    return sc_kernel


def kernel(input, matric, length, target):
    B, T, V = input.shape
    N = B * T
    inp2 = input.reshape(N, V)
    mat2 = matric.reshape(V * V, V)
    tgt = target.reshape(N).astype(jnp.int32)
    sc = _make_sc_kernel(N, T, V)
    loss_p, cnt_p = sc(inp2, mat2, length, tgt)
    return jnp.sum(loss_p) / jnp.sum(cnt_p)


# speedup vs baseline: 3.7854x; 1.2114x over previous
import functools
import math

import jax
import jax.numpy as jnp
from jax import lax
from jax.experimental import pallas as pl
from jax.experimental.pallas import tpu as pltpu
from jax.experimental.pallas import tpu_sc as plsc

BETA = 0.1
PAD = 0
LOG1MBETA = math.log(1.0 - BETA)

NC = 2
NS = 16
L = 16
NW = NC * NS

LN2 = math.log(2.0)
SQRT2 = math.sqrt(2.0)


def _log_vec(s):
    bits = lax.bitcast_convert_type(s, jnp.int32)
    e = lax.shift_right_logical(bits, 23) - 127
    m = lax.bitcast_convert_type(
        (bits & jnp.int32(0x007FFFFF)) | jnp.int32(0x3F800000), jnp.float32)
    adj = m >= SQRT2
    m = jnp.where(adj, 0.5 * m, m)
    e = e + jnp.where(adj, 1, 0)
    t = (m - 1.0) / (m + 1.0)
    t2 = t * t
    lm = 2.0 * t * (1.0 + t2 * (1.0 / 3.0 + t2 * (0.2 + t2 * (1.0 / 7.0))))
    return e.astype(jnp.float32) * LN2 + lm


def _make_sc_kernel(N, T, V):
    rows_per_w = N // NW
    blk = 80
    nblk = rows_per_w // blk
    ngrp = blk // L
    nvec = V // L

    mesh = plsc.VectorSubcoreMesh(
        core_axis_name="c", subcore_axis_name="s", num_cores=NC, num_subcores=NS)

    @functools.partial(
        pl.kernel,
        out_type=(
            jax.ShapeDtypeStruct((NW, L), jnp.float32),
            jax.ShapeDtypeStruct((NW, L), jnp.float32),
        ),
        mesh=mesh,
        compiler_params=pltpu.CompilerParams(needs_layout_passes=False),
        scratch_types=[
            pltpu.VMEM((rows_per_w,), jnp.int32),
            pltpu.VMEM((128,), jnp.float32),
            pltpu.VMEM((blk,), jnp.int32),
            pltpu.VMEM((blk,), jnp.int32),
            pltpu.VMEM((blk, V), jnp.float32),
            pltpu.VMEM((blk, V), jnp.float32),
            pltpu.VMEM((blk, V), jnp.float32),
            pltpu.VMEM((blk, V), jnp.float32),
            pltpu.VMEM((blk, L), jnp.float32),
            pltpu.VMEM((blk, L), jnp.float32),
            pltpu.VMEM((blk, L), jnp.float32),
            pltpu.VMEM((blk, L), jnp.float32),
            pltpu.VMEM((L,), jnp.float32),
            pltpu.VMEM((L,), jnp.float32),
            pltpu.SemaphoreType.DMA,
            pltpu.SemaphoreType.DMA,
            pltpu.SemaphoreType.DMA,
            pltpu.SemaphoreType.DMA,
        ],
    )
    def sc_kernel(inp_hbm, mat_hbm, len_hbm, tgt_hbm, loss_out, cnt_out,
                  tgt_v, len_v, idx0, idx1, x0, x1, m0, m1,
                  xmax_v, se_v, mdx_v, ms_v, loss_acc, cnt_acc,
                  sem_x0, sem_x1, sem_m0, sem_m1):
        wid = lax.axis_index("s") * NC + lax.axis_index("c")
        wbase = wid * rows_per_w

        pltpu.sync_copy(tgt_hbm.at[pl.ds(wbase, rows_per_w)], tgt_v)
        pltpu.sync_copy(len_hbm, len_v)

        zeros = jnp.zeros((L,), jnp.float32)
        loss_acc[...] = zeros
        cnt_acc[...] = zeros

        iota = lax.iota(jnp.int32, L)
        slots = ((idx0, x0, m0, sem_x0, sem_m0),
                 (idx1, x1, m1, sem_x1, sem_m1))

        def compute_idx(b, idx_v):
            base_l = b * blk
            for g in range(ngrp):
                pvec = base_l + g * L + iota
                cur = tgt_v[pl.ds(base_l + g * L, L)]
                prev = plsc.load_gather(tgt_v, [jnp.maximum(pvec - 1, 0)])
                forth = jnp.where(pvec % T == 0, 0, prev)
                idx_v[pl.ds(g * L, L)] = forth * V + cur

        def start_dma(b, slot):
            idx_v, x_v, m_v, sem_x, sem_m = slot
            compute_idx(b, idx_v)
            pltpu.async_copy(mat_hbm.at[idx_v], m_v, sem_m)
            pltpu.async_copy(inp_hbm.at[pl.ds(wbase + b * blk, blk)], x_v, sem_x)

        def process_block(b, sidx):
            idx_v, x_v, m_v, sem_x, sem_m = slots[sidx]
            nxt = slots[1 - sidx]

            @pl.when(b + 1 < nblk)
            def _():
                start_dma(b + 1, nxt)

            pltpu.make_async_copy(mat_hbm.at[idx_v], m_v, sem_m).wait()
            pltpu.make_async_copy(
                inp_hbm.at[pl.ds(0, blk)], x_v, sem_x).wait()

            base_l = b * blk

            def row_body(r, _):
                xv = [x_v[r, pl.ds(16 * j, L)] for j in range(nvec)]
                mx = xv[0]
                for j in range(1, nvec):
                    mx = jnp.maximum(mx, xv[j])
                xmax = jnp.max(mx)
                se = jnp.exp(xv[0] - xmax)
                mv = m_v[r, pl.ds(0, L)]
                mdx = mv * xv[0]
                ms = mv
                for j in range(1, nvec):
                    se = se + jnp.exp(xv[j] - xmax)
                    mv = m_v[r, pl.ds(16 * j, L)]
                    mdx = mdx + mv * xv[j]
                    ms = ms + mv
                xmax_v[r, :] = jnp.broadcast_to(xmax, (L,))
                se_v[r, :] = jnp.broadcast_to(jnp.sum(se), (L,))
                mdx_v[r, :] = jnp.broadcast_to(jnp.sum(mdx), (L,))
                ms_v[r, :] = jnp.broadcast_to(jnp.sum(ms), (L,))
                return 0

            lax.fori_loop(0, blk, row_body, 0, unroll=2)

            for g in range(ngrp):
                pvec = base_l + g * L + iota
                cur = tgt_v[pl.ds(base_l + g * L, L)]
                lrows = g * L + iota
                xt = plsc.load_gather(x_v, [lrows, cur])
                mt = plsc.load_gather(m_v, [lrows, cur])
                zero16 = jnp.zeros((L,), jnp.int32)
                xmax = plsc.load_gather(xmax_v, [lrows, zero16])
                se = plsc.load_gather(se_v, [lrows, zero16])
                lse = xmax + _log_vec(se)
                mdx = plsc.load_gather(mdx_v, [lrows, zero16])
                ms = plsc.load_gather(ms_v, [lrows, zero16])
                bidx = wid * (rows_per_w // T) + pvec // T
                lens = plsc.load_gather(len_v, [bidx])
                s = 1.0 - jnp.exp(LOG1MBETA / lens)
                c = s * (mdx - ms * lse) + (1.0 - s * ms - s * mt) * (xt - lse)
                ispad = cur == PAD
                loss_acc[...] = loss_acc[...] + jnp.where(ispad, 0.0, -c)
                cnt_acc[...] = cnt_acc[...] + jnp.where(ispad, 1.0, 0.0)

        start_dma(0, slots[0])

        def pair_body(i, _):
            process_block(2 * i, 0)
            process_block(2 * i + 1, 1)
            return 0

        lax.fori_loop(0, nblk // 2, pair_body, 0)

        pltpu.sync_copy(loss_acc, loss_out.at[wid])
        pltpu.sync_copy(cnt_acc, cnt_out.at[wid])

    return sc_kernel


def kernel(input, matric, length, target):
    B, T, V = input.shape
    N = B * T
    inp2 = input.reshape(N, V)
    mat2 = matric.reshape(V * V, V)
    tgt = target.reshape(N).astype(jnp.int32)
    sc = _make_sc_kernel(N, T, V)
    loss_p, cnt_p = sc(inp2, mat2, length, tgt)
    return jnp.sum(loss_p) / jnp.sum(cnt_p)
